# Initial kernel scaffold; baseline (speedup 1.0000x reference)
import jax
import jax.numpy as jnp
from jax.experimental import pallas as pl


def kernel(x, edge_index, W1, b1, W2, b2, W3, b3, g1w, g1b, g2w, g2b, g3w, g3b, Wjk, bjk, Wf1, bf1, Wf2, bf2):
    raise NotImplementedError("write your pallas kernel here")


# speedup vs baseline: 2.5022x; 2.5022x over previous
import jax
import jax.numpy as jnp
from jax.experimental import pallas as pl


def _head_body(cat_ref, wjk_ref, bjk_ref, wf1_ref, bf1_ref, wf2_ref, bf2_ref, out_ref):
    cat = cat_ref[...]
    h = cat @ wjk_ref[...] + bjk_ref[...]
    h = jnp.tanh(h @ wf1_ref[...] + bf1_ref[...])
    out_ref[...] = h @ wf2_ref[...] + bf2_ref[...]


def kernel(x, edge_index, W1, b1, W2, b2, W3, b3, g1w, g1b, g2w, g2b, g3w, g3b,
           Wjk, bjk, Wf1, bf1, Wf2, bf2):
    n = x.shape[0]
    row = edge_index[0]
    col = edge_index[1]
    ones = jnp.ones(row.shape[0], dtype=x.dtype)
    deg = jax.ops.segment_sum(ones, col, num_segments=n) + 1.0
    dinv = deg ** -0.5

    def conv(h, W, b):
        u = (h @ W) * dinv[:, None]
        acc = jax.ops.segment_sum(u[row], col, num_segments=n) + u
        return acc * dinv[:, None] + b

    def graph_layernorm(h, w, bb, eps=1e-5):
        m = jnp.mean(h)
        std = jnp.sqrt(jnp.mean((h - m) ** 2))
        return ((h - m) / (std + eps)) * w + bb

    lasts = []
    h = x
    for (W, b, gw, gb) in ((W1, b1, g1w, g1b), (W2, b2, g2w, g2b), (W3, b3, g3w, g3b)):
        h = conv(h, W, b)
        h = graph_layernorm(h, gw, gb)
        h = jnp.tanh(h)
        lasts.append(h[-1:])
    cat = jnp.concatenate(lasts, axis=-1)
    out = pl.pallas_call(
        _head_body,
        out_shape=jax.ShapeDtypeStruct((1, 1), jnp.float32),
    )(cat, Wjk, bjk.reshape(1, -1), Wf1, bf1.reshape(1, -1), Wf2, bf2.reshape(1, -1))
    return out[0]


# speedup vs baseline: 14.0477x; 5.6142x over previous
import functools

import jax
import jax.numpy as jnp
from jax import lax
from jax.experimental import pallas as pl
from jax.experimental.pallas import tpu as pltpu
from jax.experimental.pallas import tpu_sc as plsc

N, E, H = 10000, 320000, 128
HH = H // 2
NC, NS, L = 2, 16, 16
NP = 10240
RPT = NP // NS
G = 80
BM = 256
NB = NP // BM
LAST = N - 1
EPS = 1e-5

_mesh = plsc.VectorSubcoreMesh(core_axis_name="c", subcore_axis_name="s")


def _sc_deg_body(col_hbm, degp_hbm, colbuf, cchunk, onesbuf, zbuf, deg_sp, sem):
    del sem
    c = lax.axis_index("c")
    s = lax.axis_index("s")
    ec = E // (NC * NS)
    for k in range(RPT // L):
        zbuf[pl.ds(k * L, L)] = jnp.zeros((L,), jnp.float32)
    for k in range(G // L):
        onesbuf[pl.ds(k * L, L)] = jnp.ones((L,), jnp.float32)
    pltpu.sync_copy(zbuf, deg_sp.at[pl.ds(s * RPT, RPT)])
    base = (c * NS + s) * ec
    pltpu.sync_copy(col_hbm.at[pl.ds(base, ec)], colbuf)
    plsc.subcore_barrier()

    def chunk(i, carry):
        for j in range(G // L):
            cchunk[pl.ds(j * L, L)] = colbuf[pl.ds(i * G + j * L, L)]
        pltpu.sync_copy(onesbuf, deg_sp.at[cchunk], add=True)
        return carry

    lax.fori_loop(0, ec // G, chunk, 0)
    plsc.subcore_barrier()
    pltpu.sync_copy(deg_sp.at[pl.ds(s * RPT, RPT)],
                    degp_hbm.at[c, pl.ds(s * RPT, RPT)])


_sc_deg = pl.kernel(
    _sc_deg_body,
    out_type=jax.ShapeDtypeStruct((NC, NP), jnp.float32),
    mesh=_mesh,
    scratch_types=[
        pltpu.VMEM((E // (NC * NS),), jnp.int32),
        pltpu.VMEM((G,), jnp.int32),
        pltpu.VMEM((G,), jnp.float32),
        pltpu.VMEM((RPT,), jnp.float32),
        pltpu.VMEM_SHARED((NP,), jnp.float32),
        pltpu.SemaphoreType.DMA,
    ],
)


def _sc_edge_body(u_hbm, row_hbm, col_hbm, out_hbm, rowbuf, colbuf, cchunk,
                  gbuf, acc_sp, sem):
    c = lax.axis_index("c")
    s = lax.axis_index("s")
    et = E // (NC * NS)
    pltpu.sync_copy(u_hbm.at[pl.ds(s * RPT, RPT)],
                    acc_sp.at[pl.ds(s * RPT, RPT)])
    ebase = (c * NS + s) * et
    pltpu.sync_copy(row_hbm.at[pl.ds(ebase, et)], rowbuf)
    pltpu.sync_copy(col_hbm.at[pl.ds(ebase, et)], colbuf)
    plsc.subcore_barrier()

    def chunk(i, carry):
        for j in range(G // L):
            cchunk[pl.ds(j * L, L)] = colbuf[pl.ds(i * G + j * L, L)]
        pltpu.async_copy(u_hbm.at[rowbuf.at[pl.ds(i * G, G)]], gbuf, sem).wait()
        pltpu.sync_copy(gbuf, acc_sp.at[cchunk], add=True)
        return carry

    lax.fori_loop(0, et // G, chunk, 0)
    plsc.subcore_barrier()
    pltpu.sync_copy(acc_sp.at[pl.ds(s * RPT, RPT)],
                    out_hbm.at[pl.ds(c * NP + s * RPT, RPT)])


_sc_edge = pl.kernel(
    _sc_edge_body,
    out_type=jax.ShapeDtypeStruct((NC * NP, H), jnp.float32),
    mesh=_mesh,
    scratch_types=[
        pltpu.VMEM((E // (NC * NS),), jnp.int32),
        pltpu.VMEM((E // (NC * NS),), jnp.int32),
        pltpu.VMEM((G,), jnp.int32),
        pltpu.VMEM((G, H), jnp.float32),
        pltpu.VMEM_SHARED((NP, H), jnp.float32),
        pltpu.SemaphoreType.DMA,
    ],
)


def _mm_scale_body(x_ref, w_ref, dinv_ref, out_ref):
    out_ref[...] = jnp.dot(x_ref[...], w_ref[...],
                           preferred_element_type=jnp.float32) * dinv_ref[...]


def _mm_scale(xp, w, dinv1):
    return pl.pallas_call(
        _mm_scale_body,
        grid=(NB,),
        in_specs=[
            pl.BlockSpec((BM, H), lambda i: (i, 0)),
            pl.BlockSpec((H, H), lambda i: (0, 0)),
            pl.BlockSpec((BM, 1), lambda i: (i, 0)),
        ],
        out_specs=pl.BlockSpec((BM, H), lambda i: (i, 0)),
        out_shape=jax.ShapeDtypeStruct((NP, H), jnp.float32),
    )(xp, w, dinv1)


def _combine_body(acc_ref, u_ref, dinv_ref, b_ref, y_ref, stats_ref, sums_ref):
    i = pl.program_id(0)
    y = (acc_ref[0] + acc_ref[1] - u_ref[...]) * dinv_ref[...] + b_ref[...]
    y_ref[...] = y
    ridx = lax.broadcasted_iota(jnp.int32, (BM, H), 0) + i * BM
    ym = jnp.where(ridx < N, y, 0.0)

    @pl.when(i == 0)
    def _():
        sums_ref[0] = 0.0
        sums_ref[1] = 0.0

    sums_ref[0] += jnp.sum(ym)
    sums_ref[1] += jnp.sum(ym * ym)

    @pl.when(i == NB - 1)
    def _():
        stats_ref[0] = sums_ref[0]
        stats_ref[1] = sums_ref[1]


def _combine(acc, u, dinv1, b):
    return pl.pallas_call(
        _combine_body,
        grid=(NB,),
        in_specs=[
            pl.BlockSpec((NC, BM, H), lambda i: (0, i, 0)),
            pl.BlockSpec((BM, H), lambda i: (i, 0)),
            pl.BlockSpec((BM, 1), lambda i: (i, 0)),
            pl.BlockSpec((1, H), lambda i: (0, 0)),
        ],
        out_specs=[
            pl.BlockSpec((BM, H), lambda i: (i, 0)),
            pl.BlockSpec(memory_space=pltpu.SMEM),
        ],
        out_shape=[
            jax.ShapeDtypeStruct((NP, H), jnp.float32),
            jax.ShapeDtypeStruct((2,), jnp.float32),
        ],
        scratch_shapes=[pltpu.SMEM((2,), jnp.float32)],
    )(acc, u, dinv1, b.reshape(1, H))


def _norm_mm_body(y_ref, scl_ref, gw_ref, gb_ref, dinv_ref, w_ref,
                  out_ref, tl_ref):
    m = scl_ref[0]
    std = scl_ref[1]
    t = jnp.tanh((y_ref[...] - m) / (std + EPS) * gw_ref[...] + gb_ref[...])
    out_ref[...] = jnp.dot(t, w_ref[...],
                           preferred_element_type=jnp.float32) * dinv_ref[...]
    tl_ref[...] = t[LAST % BM:LAST % BM + 1, :]


def _norm_mm(y, scl, gw, gb, dinv1, w):
    return pl.pallas_call(
        _norm_mm_body,
        grid=(NB,),
        in_specs=[
            pl.BlockSpec((BM, H), lambda i: (i, 0)),
            pl.BlockSpec(memory_space=pltpu.SMEM),
            pl.BlockSpec((1, H), lambda i: (0, 0)),
            pl.BlockSpec((1, H), lambda i: (0, 0)),
            pl.BlockSpec((BM, 1), lambda i: (i, 0)),
            pl.BlockSpec((H, H), lambda i: (0, 0)),
        ],
        out_specs=[
            pl.BlockSpec((BM, H), lambda i: (i, 0)),
            pl.BlockSpec((1, H), lambda i: (0, 0)),
        ],
        out_shape=[
            jax.ShapeDtypeStruct((NP, H), jnp.float32),
            jax.ShapeDtypeStruct((1, H), jnp.float32),
        ],
    )(y, scl, gw.reshape(1, H), gb.reshape(1, H), dinv1, w)


def _head_body(y_ref, scl_ref, g3w_ref, g3b_ref, t1_ref, t2_ref, wjk_ref,
               bjk_ref, wf1_ref, bf1_ref, wf2_ref, bf2_ref, out_ref):
    m = scl_ref[0]
    std = scl_ref[1]
    r0 = LAST % BM
    t3 = jnp.tanh((y_ref[r0:r0 + 1, :] - m) / (std + EPS) * g3w_ref[...]
                  + g3b_ref[...])
    cat = jnp.concatenate([t1_ref[...], t2_ref[...], t3], axis=1)
    h = jnp.dot(cat, wjk_ref[...], preferred_element_type=jnp.float32) \
        + bjk_ref[...]
    h = jnp.tanh(jnp.dot(h, wf1_ref[...], preferred_element_type=jnp.float32)
                 + bf1_ref[...])
    out_ref[...] = jnp.dot(h, wf2_ref[...],
                           preferred_element_type=jnp.float32) + bf2_ref[...]


def _head(y3, scl3, g3w, g3b, t1, t2, wjk, bjk, wf1, bf1, wf2p, bf2p):
    return pl.pallas_call(
        _head_body,
        in_specs=[
            pl.BlockSpec((BM, H), lambda: (0, 0)),
            pl.BlockSpec(memory_space=pltpu.SMEM),
            pl.BlockSpec((1, H), lambda: (0, 0)),
            pl.BlockSpec((1, H), lambda: (0, 0)),
            pl.BlockSpec((1, H), lambda: (0, 0)),
            pl.BlockSpec((1, H), lambda: (0, 0)),
            pl.BlockSpec((3 * H, H), lambda: (0, 0)),
            pl.BlockSpec((1, H), lambda: (0, 0)),
            pl.BlockSpec((H, HH), lambda: (0, 0)),
            pl.BlockSpec((1, HH), lambda: (0, 0)),
            pl.BlockSpec((HH, H), lambda: (0, 0)),
            pl.BlockSpec((1, H), lambda: (0, 0)),
        ],
        out_specs=pl.BlockSpec((1, H), lambda: (0, 0)),
        out_shape=jax.ShapeDtypeStruct((1, H), jnp.float32),
    )(y3[(LAST // BM) * BM:(LAST // BM + 1) * BM], scl3, g3w.reshape(1, H),
      g3b.reshape(1, H), t1, t2, wjk, bjk.reshape(1, H), wf1,
      bf1.reshape(1, HH), wf2p, bf2p)


def kernel(x, edge_index, W1, b1, W2, b2, W3, b3, g1w, g1b, g2w, g2b, g3w, g3b,
           Wjk, bjk, Wf1, bf1, Wf2, bf2):
    row = edge_index[0]
    col = edge_index[1]
    xp = jnp.pad(x, ((0, NP - N), (0, 0)))

    degp = _sc_deg(col)
    dinv1 = lax.rsqrt(degp[0] + degp[1] + 1.0).reshape(NP, 1)

    nh = float(N * H)

    def stats_to_scl(stats):
        m = stats[0] / nh
        std = jnp.sqrt(jnp.maximum(stats[1] / nh - m * m, 0.0))
        return jnp.stack([m, std])

    u = _mm_scale(xp, W1, dinv1)
    tls = []
    ys = None
    for li, (b, gw, gb, wn) in enumerate(((b1, g1w, g1b, W2),
                                          (b2, g2w, g2b, W3),
                                          (b3, g3w, g3b, None))):
        acc = _sc_edge(u, row, col).reshape(NC, NP, H)
        y, stats = _combine(acc, u, dinv1, b)
        scl = stats_to_scl(stats)
        if wn is not None:
            u, tl = _norm_mm(y, scl, gw, gb, dinv1, wn)
            tls.append(tl)
        else:
            ys = (y, scl)

    wf2p = jnp.pad(Wf2, ((0, 0), (0, H - 1)))
    bf2p = jnp.pad(bf2.reshape(1, 1), ((0, 0), (0, H - 1)))
    out = _head(ys[0], ys[1], g3w, g3b, tls[0], tls[1], Wjk, bjk, Wf1, bf1,
                wf2p, bf2p)
    return out[0, :1]


# speedup vs baseline: 17.2932x; 1.2310x over previous
import functools

import jax
import jax.numpy as jnp
from jax import lax
from jax.experimental import pallas as pl
from jax.experimental.pallas import tpu as pltpu
from jax.experimental.pallas import tpu_sc as plsc

N, E, H = 10000, 320000, 128
HH = H // 2
NC, NS, L = 2, 16, 16
NP = 10240
RPT = NP // NS
G = 80
BM = 256
NB = NP // BM
LAST = N - 1
EPS = 1e-5

_mesh = plsc.VectorSubcoreMesh(core_axis_name="c", subcore_axis_name="s")


def _sc_deg_body(col_hbm, degp_hbm, colbuf, cchunk, onesbuf, zbuf, deg_sp, sem):
    del sem
    c = lax.axis_index("c")
    s = lax.axis_index("s")
    ec = E // (NC * NS)
    for k in range(RPT // L):
        zbuf[pl.ds(k * L, L)] = jnp.zeros((L,), jnp.float32)
    for k in range(G // L):
        onesbuf[pl.ds(k * L, L)] = jnp.ones((L,), jnp.float32)
    pltpu.sync_copy(zbuf, deg_sp.at[pl.ds(s * RPT, RPT)])
    base = (c * NS + s) * ec
    pltpu.sync_copy(col_hbm.at[pl.ds(base, ec)], colbuf)
    plsc.subcore_barrier()

    def chunk(i, carry):
        for j in range(G // L):
            cchunk[pl.ds(j * L, L)] = colbuf[pl.ds(i * G + j * L, L)]
        pltpu.sync_copy(onesbuf, deg_sp.at[cchunk], add=True)
        return carry

    lax.fori_loop(0, ec // G, chunk, 0)
    plsc.subcore_barrier()
    pltpu.sync_copy(deg_sp.at[pl.ds(s * RPT, RPT)],
                    degp_hbm.at[c, pl.ds(s * RPT, RPT)])


_sc_deg = pl.kernel(
    _sc_deg_body,
    out_type=jax.ShapeDtypeStruct((NC, NP), jnp.float32),
    mesh=_mesh,
    scratch_types=[
        pltpu.VMEM((E // (NC * NS),), jnp.int32),
        pltpu.VMEM((G,), jnp.int32),
        pltpu.VMEM((G,), jnp.float32),
        pltpu.VMEM((RPT,), jnp.float32),
        pltpu.VMEM_SHARED((NP,), jnp.float32),
        pltpu.SemaphoreType.DMA,
    ],
)


NBUF = 2


def _sc_edge_body(u_hbm, row_hbm, col_hbm, out_hbm, rowbuf, colbuf, cchunk,
                  gbuf, acc_sp, gsem0, gsem1, ssem0, ssem1):
    c = lax.axis_index("c")
    s = lax.axis_index("s")
    gsems = (gsem0, gsem1)
    ssems = (ssem0, ssem1)
    et = E // (NC * NS)
    pltpu.sync_copy(u_hbm.at[pl.ds(s * RPT, RPT)],
                    acc_sp.at[pl.ds(s * RPT, RPT)])
    ebase = (c * NS + s) * et
    pltpu.sync_copy(row_hbm.at[pl.ds(ebase, et)], rowbuf)
    pltpu.sync_copy(col_hbm.at[pl.ds(ebase, et)], colbuf)
    plsc.subcore_barrier()

    def scat_desc(k):
        return pltpu.make_async_copy(gbuf.at[k], acc_sp.at[cchunk.at[k]],
                                     ssems[k])

    def start_gather(i, k):
        for j in range(G // L):
            cchunk[k, pl.ds(j * L, L)] = colbuf[pl.ds(i * G + j * L, L)]
        return pltpu.async_copy(
            u_hbm.at[rowbuf.at[pl.ds(i * G, G)]], gbuf.at[k], gsems[k])

    nch = et // G
    ngrp, rem = divmod(nch, NBUF)

    def group(q, carry):
        gds = []
        for k in range(NBUF):
            @pl.when(q > 0)
            def _():
                scat_desc(k).wait()
            gds.append(start_gather(q * NBUF + k, k))
        for k in range(NBUF):
            gds[k].wait()
            pltpu.async_copy(gbuf.at[k], acc_sp.at[cchunk.at[k]], ssems[k],
                             add=True)
        return carry

    lax.fori_loop(0, ngrp, group, 0)
    for k in range(rem):
        if ngrp > 0:
            scat_desc(k).wait()
        gd = start_gather(ngrp * NBUF + k, k)
        gd.wait()
        pltpu.async_copy(gbuf.at[k], acc_sp.at[cchunk.at[k]], ssems[k],
                         add=True)
    for k in range(NBUF):
        if ngrp > 0 or k < rem:
            scat_desc(k).wait()
    plsc.subcore_barrier()
    pltpu.sync_copy(acc_sp.at[pl.ds(s * RPT, RPT)],
                    out_hbm.at[pl.ds(c * NP + s * RPT, RPT)])


_sc_edge = pl.kernel(
    _sc_edge_body,
    out_type=jax.ShapeDtypeStruct((NC * NP, H), jnp.float32),
    mesh=_mesh,
    scratch_types=[
        pltpu.VMEM((E // (NC * NS),), jnp.int32),
        pltpu.VMEM((E // (NC * NS),), jnp.int32),
        pltpu.VMEM((NBUF, G), jnp.int32),
        pltpu.VMEM((NBUF, G, H), jnp.float32),
        pltpu.VMEM_SHARED((NP, H), jnp.float32),
        pltpu.SemaphoreType.DMA,
        pltpu.SemaphoreType.DMA,
        pltpu.SemaphoreType.DMA,
        pltpu.SemaphoreType.DMA,
    ],
)


def _mm_scale_body(x_ref, w_ref, dinv_ref, out_ref):
    out_ref[...] = jnp.dot(x_ref[...], w_ref[...],
                           preferred_element_type=jnp.float32) * dinv_ref[...]


def _mm_scale(xp, w, dinv1):
    return pl.pallas_call(
        _mm_scale_body,
        grid=(NB,),
        in_specs=[
            pl.BlockSpec((BM, H), lambda i: (i, 0)),
            pl.BlockSpec((H, H), lambda i: (0, 0)),
            pl.BlockSpec((BM, 1), lambda i: (i, 0)),
        ],
        out_specs=pl.BlockSpec((BM, H), lambda i: (i, 0)),
        out_shape=jax.ShapeDtypeStruct((NP, H), jnp.float32),
    )(xp, w, dinv1)


def _combine_body(acc_ref, u_ref, dinv_ref, b_ref, y_ref, stats_ref, sums_ref):
    i = pl.program_id(0)
    y = (acc_ref[0] + acc_ref[1] - u_ref[...]) * dinv_ref[...] + b_ref[...]
    y_ref[...] = y
    ridx = lax.broadcasted_iota(jnp.int32, (BM, H), 0) + i * BM
    ym = jnp.where(ridx < N, y, 0.0)

    @pl.when(i == 0)
    def _():
        sums_ref[0] = 0.0
        sums_ref[1] = 0.0

    sums_ref[0] += jnp.sum(ym)
    sums_ref[1] += jnp.sum(ym * ym)

    @pl.when(i == NB - 1)
    def _():
        stats_ref[0] = sums_ref[0]
        stats_ref[1] = sums_ref[1]


def _combine(acc, u, dinv1, b):
    return pl.pallas_call(
        _combine_body,
        grid=(NB,),
        in_specs=[
            pl.BlockSpec((NC, BM, H), lambda i: (0, i, 0)),
            pl.BlockSpec((BM, H), lambda i: (i, 0)),
            pl.BlockSpec((BM, 1), lambda i: (i, 0)),
            pl.BlockSpec((1, H), lambda i: (0, 0)),
        ],
        out_specs=[
            pl.BlockSpec((BM, H), lambda i: (i, 0)),
            pl.BlockSpec(memory_space=pltpu.SMEM),
        ],
        out_shape=[
            jax.ShapeDtypeStruct((NP, H), jnp.float32),
            jax.ShapeDtypeStruct((2,), jnp.float32),
        ],
        scratch_shapes=[pltpu.SMEM((2,), jnp.float32)],
    )(acc, u, dinv1, b.reshape(1, H))


def _norm_mm_body(y_ref, scl_ref, gw_ref, gb_ref, dinv_ref, w_ref,
                  out_ref, tl_ref):
    m = scl_ref[0]
    std = scl_ref[1]
    t = jnp.tanh((y_ref[...] - m) / (std + EPS) * gw_ref[...] + gb_ref[...])
    out_ref[...] = jnp.dot(t, w_ref[...],
                           preferred_element_type=jnp.float32) * dinv_ref[...]
    tl_ref[...] = t[LAST % BM:LAST % BM + 1, :]


def _norm_mm(y, scl, gw, gb, dinv1, w):
    return pl.pallas_call(
        _norm_mm_body,
        grid=(NB,),
        in_specs=[
            pl.BlockSpec((BM, H), lambda i: (i, 0)),
            pl.BlockSpec(memory_space=pltpu.SMEM),
            pl.BlockSpec((1, H), lambda i: (0, 0)),
            pl.BlockSpec((1, H), lambda i: (0, 0)),
            pl.BlockSpec((BM, 1), lambda i: (i, 0)),
            pl.BlockSpec((H, H), lambda i: (0, 0)),
        ],
        out_specs=[
            pl.BlockSpec((BM, H), lambda i: (i, 0)),
            pl.BlockSpec((1, H), lambda i: (0, 0)),
        ],
        out_shape=[
            jax.ShapeDtypeStruct((NP, H), jnp.float32),
            jax.ShapeDtypeStruct((1, H), jnp.float32),
        ],
    )(y, scl, gw.reshape(1, H), gb.reshape(1, H), dinv1, w)


def _head_body(y_ref, scl_ref, g3w_ref, g3b_ref, t1_ref, t2_ref, wjk_ref,
               bjk_ref, wf1_ref, bf1_ref, wf2_ref, bf2_ref, out_ref):
    m = scl_ref[0]
    std = scl_ref[1]
    r0 = LAST % BM
    t3 = jnp.tanh((y_ref[r0:r0 + 1, :] - m) / (std + EPS) * g3w_ref[...]
                  + g3b_ref[...])
    cat = jnp.concatenate([t1_ref[...], t2_ref[...], t3], axis=1)
    h = jnp.dot(cat, wjk_ref[...], preferred_element_type=jnp.float32) \
        + bjk_ref[...]
    h = jnp.tanh(jnp.dot(h, wf1_ref[...], preferred_element_type=jnp.float32)
                 + bf1_ref[...])
    out_ref[...] = jnp.dot(h, wf2_ref[...],
                           preferred_element_type=jnp.float32) + bf2_ref[...]


def _head(y3, scl3, g3w, g3b, t1, t2, wjk, bjk, wf1, bf1, wf2p, bf2p):
    return pl.pallas_call(
        _head_body,
        in_specs=[
            pl.BlockSpec((BM, H), lambda: (0, 0)),
            pl.BlockSpec(memory_space=pltpu.SMEM),
            pl.BlockSpec((1, H), lambda: (0, 0)),
            pl.BlockSpec((1, H), lambda: (0, 0)),
            pl.BlockSpec((1, H), lambda: (0, 0)),
            pl.BlockSpec((1, H), lambda: (0, 0)),
            pl.BlockSpec((3 * H, H), lambda: (0, 0)),
            pl.BlockSpec((1, H), lambda: (0, 0)),
            pl.BlockSpec((H, HH), lambda: (0, 0)),
            pl.BlockSpec((1, HH), lambda: (0, 0)),
            pl.BlockSpec((HH, H), lambda: (0, 0)),
            pl.BlockSpec((1, H), lambda: (0, 0)),
        ],
        out_specs=pl.BlockSpec((1, H), lambda: (0, 0)),
        out_shape=jax.ShapeDtypeStruct((1, H), jnp.float32),
    )(y3[(LAST // BM) * BM:(LAST // BM + 1) * BM], scl3, g3w.reshape(1, H),
      g3b.reshape(1, H), t1, t2, wjk, bjk.reshape(1, H), wf1,
      bf1.reshape(1, HH), wf2p, bf2p)


def kernel(x, edge_index, W1, b1, W2, b2, W3, b3, g1w, g1b, g2w, g2b, g3w, g3b,
           Wjk, bjk, Wf1, bf1, Wf2, bf2):
    row = edge_index[0]
    col = edge_index[1]
    xp = jnp.pad(x, ((0, NP - N), (0, 0)))

    degp = _sc_deg(col)
    dinv1 = lax.rsqrt(degp[0] + degp[1] + 1.0).reshape(NP, 1)

    nh = float(N * H)

    def stats_to_scl(stats):
        m = stats[0] / nh
        std = jnp.sqrt(jnp.maximum(stats[1] / nh - m * m, 0.0))
        return jnp.stack([m, std])

    u = _mm_scale(xp, W1, dinv1)
    tls = []
    ys = None
    for li, (b, gw, gb, wn) in enumerate(((b1, g1w, g1b, W2),
                                          (b2, g2w, g2b, W3),
                                          (b3, g3w, g3b, None))):
        acc = _sc_edge(u, row, col).reshape(NC, NP, H)
        y, stats = _combine(acc, u, dinv1, b)
        scl = stats_to_scl(stats)
        if wn is not None:
            u, tl = _norm_mm(y, scl, gw, gb, dinv1, wn)
            tls.append(tl)
        else:
            ys = (y, scl)

    wf2p = jnp.pad(Wf2, ((0, 0), (0, H - 1)))
    bf2p = jnp.pad(bf2.reshape(1, 1), ((0, 0), (0, H - 1)))
    out = _head(ys[0], ys[1], g3w, g3b, tls[0], tls[1], Wjk, bjk, Wf1, bf1,
                wf2p, bf2p)
    return out[0, :1]


# speedup vs baseline: 17.8404x; 1.0316x over previous
import functools

import jax
import jax.numpy as jnp
from jax import lax
from jax.experimental import pallas as pl
from jax.experimental.pallas import tpu as pltpu
from jax.experimental.pallas import tpu_sc as plsc

N, E, H = 10000, 320000, 128
HH = H // 2
NC, NS, L = 2, 16, 16
NP = 10240
RPT = NP // NS
G = 80
BM = 256
NB = NP // BM
LAST = N - 1
EPS = 1e-5

_mesh = plsc.VectorSubcoreMesh(core_axis_name="c", subcore_axis_name="s")


def _sc_deg_body(col_hbm, degp_hbm, colbuf, cchunk, onesbuf, zbuf, deg_sp, sem):
    del sem
    c = lax.axis_index("c")
    s = lax.axis_index("s")
    ec = E // (NC * NS)
    for k in range(RPT // L):
        zbuf[pl.ds(k * L, L)] = jnp.zeros((L,), jnp.float32)
    for k in range(G // L):
        onesbuf[pl.ds(k * L, L)] = jnp.ones((L,), jnp.float32)
    pltpu.sync_copy(zbuf, deg_sp.at[pl.ds(s * RPT, RPT)])
    base = (c * NS + s) * ec
    pltpu.sync_copy(col_hbm.at[pl.ds(base, ec)], colbuf)
    plsc.subcore_barrier()

    def chunk(i, carry):
        for j in range(G // L):
            cchunk[pl.ds(j * L, L)] = colbuf[pl.ds(i * G + j * L, L)]
        pltpu.sync_copy(onesbuf, deg_sp.at[cchunk], add=True)
        return carry

    lax.fori_loop(0, ec // G, chunk, 0)
    plsc.subcore_barrier()
    pltpu.sync_copy(deg_sp.at[pl.ds(s * RPT, RPT)],
                    degp_hbm.at[c, pl.ds(s * RPT, RPT)])


_sc_deg = pl.kernel(
    _sc_deg_body,
    out_type=jax.ShapeDtypeStruct((NC, NP), jnp.float32),
    mesh=_mesh,
    scratch_types=[
        pltpu.VMEM((E // (NC * NS),), jnp.int32),
        pltpu.VMEM((G,), jnp.int32),
        pltpu.VMEM((G,), jnp.float32),
        pltpu.VMEM((RPT,), jnp.float32),
        pltpu.VMEM_SHARED((NP,), jnp.float32),
        pltpu.SemaphoreType.DMA,
    ],
)


NBUF = 2


def _sc_edge_body(u_hbm, row_hbm, col_hbm, out_hbm, rowbuf, colbuf, cchunk,
                  gbuf, acc_sp, gsem0, gsem1, ssem0, ssem1):
    c = lax.axis_index("c")
    s = lax.axis_index("s")
    gsems = (gsem0, gsem1)
    ssems = (ssem0, ssem1)
    et = E // (NC * NS)
    pltpu.sync_copy(u_hbm.at[pl.ds(s * RPT, RPT)],
                    acc_sp.at[pl.ds(s * RPT, RPT)])
    ebase = (c * NS + s) * et
    pltpu.sync_copy(row_hbm.at[pl.ds(ebase, et)], rowbuf)
    pltpu.sync_copy(col_hbm.at[pl.ds(ebase, et)], colbuf)
    plsc.subcore_barrier()

    def scat_desc(k):
        return pltpu.make_async_copy(gbuf.at[k], acc_sp.at[cchunk.at[k]],
                                     ssems[k])

    def start_gather(i, k):
        for j in range(G // L):
            cchunk[k, pl.ds(j * L, L)] = colbuf[pl.ds(i * G + j * L, L)]
        return pltpu.async_copy(
            u_hbm.at[rowbuf.at[pl.ds(i * G, G)]], gbuf.at[k], gsems[k])

    nch = et // G
    ngrp, rem = divmod(nch, NBUF)

    def group(q, carry):
        gds = []
        for k in range(NBUF):
            @pl.when(q > 0)
            def _():
                scat_desc(k).wait()
            gds.append(start_gather(q * NBUF + k, k))
        for k in range(NBUF):
            gds[k].wait()
            pltpu.async_copy(gbuf.at[k], acc_sp.at[cchunk.at[k]], ssems[k],
                             add=True)
        return carry

    lax.fori_loop(0, ngrp, group, 0)
    for k in range(rem):
        if ngrp > 0:
            scat_desc(k).wait()
        gd = start_gather(ngrp * NBUF + k, k)
        gd.wait()
        pltpu.async_copy(gbuf.at[k], acc_sp.at[cchunk.at[k]], ssems[k],
                         add=True)
    for k in range(NBUF):
        if ngrp > 0 or k < rem:
            scat_desc(k).wait()
    plsc.subcore_barrier()
    pltpu.sync_copy(acc_sp.at[pl.ds(s * RPT, RPT)],
                    out_hbm.at[pl.ds(c * NP + s * RPT, RPT)])


_sc_edge = pl.kernel(
    _sc_edge_body,
    out_type=jax.ShapeDtypeStruct((NC * NP, H), jnp.float32),
    mesh=_mesh,
    scratch_types=[
        pltpu.VMEM((E // (NC * NS),), jnp.int32),
        pltpu.VMEM((E // (NC * NS),), jnp.int32),
        pltpu.VMEM((NBUF, G), jnp.int32),
        pltpu.VMEM((NBUF, G, H), jnp.float32),
        pltpu.VMEM_SHARED((NP, H), jnp.float32),
        pltpu.SemaphoreType.DMA,
        pltpu.SemaphoreType.DMA,
        pltpu.SemaphoreType.DMA,
        pltpu.SemaphoreType.DMA,
    ],
)


def _mm_scale_body(x_ref, w_ref, dinv_ref, out_ref):
    out_ref[...] = jnp.dot(x_ref[...], w_ref[...],
                           preferred_element_type=jnp.float32) * dinv_ref[...]


def _mm_scale(xp, w, dinv1):
    return pl.pallas_call(
        _mm_scale_body,
        grid=(NB,),
        in_specs=[
            pl.BlockSpec((BM, H), lambda i: (i, 0)),
            pl.BlockSpec((H, H), lambda i: (0, 0)),
            pl.BlockSpec((BM, 1), lambda i: (i, 0)),
        ],
        out_specs=pl.BlockSpec((BM, H), lambda i: (i, 0)),
        out_shape=jax.ShapeDtypeStruct((NP, H), jnp.float32),
    )(xp, w, dinv1)


NH = float(N * H)


def _stats_accum(i, y, sums_ref):
    ridx = lax.broadcasted_iota(jnp.int32, (BM, H), 0) + i * BM
    ym = jnp.where(ridx < N, y, 0.0)

    @pl.when(i == 0)
    def _():
        sums_ref[0] = 0.0
        sums_ref[1] = 0.0

    sums_ref[0] += jnp.sum(ym)
    sums_ref[1] += jnp.sum(ym * ym)


def _mstd(sums_ref):
    m = sums_ref[0] / NH
    std = jnp.sqrt(jnp.maximum(sums_ref[1] / NH - m * m, 0.0))
    return m, std


def _fused_body(acc_ref, u_ref, dinv_ref, b_ref, gw_ref, gb_ref, w_ref,
                out_ref, tl_ref, ybuf, sums_ref):
    p = pl.program_id(0)
    i = pl.program_id(1)

    @pl.when(p == 0)
    def _():
        y = (acc_ref[0] + acc_ref[1] - u_ref[...]) * dinv_ref[...] + b_ref[...]
        ybuf[pl.ds(i * BM, BM), :] = y
        _stats_accum(i, y, sums_ref)

    @pl.when(p == 1)
    def _():
        m, std = _mstd(sums_ref)
        t = jnp.tanh((ybuf[pl.ds(i * BM, BM), :] - m) / (std + EPS)
                     * gw_ref[...] + gb_ref[...])
        out_ref[...] = jnp.dot(
            t, w_ref[...], preferred_element_type=jnp.float32) * dinv_ref[...]
        tl_ref[...] = t[LAST % BM:LAST % BM + 1, :]


def _fused(acc, u, dinv1, b, gw, gb, wn):
    return pl.pallas_call(
        _fused_body,
        grid=(2, NB),
        in_specs=[
            pl.BlockSpec((NC, BM, H), lambda p, i: (0, (1 - p) * i, 0)),
            pl.BlockSpec((BM, H), lambda p, i: ((1 - p) * i, 0)),
            pl.BlockSpec((BM, 1), lambda p, i: (i, 0)),
            pl.BlockSpec((1, H), lambda p, i: (0, 0)),
            pl.BlockSpec((1, H), lambda p, i: (0, 0)),
            pl.BlockSpec((1, H), lambda p, i: (0, 0)),
            pl.BlockSpec((H, H), lambda p, i: (0, 0)),
        ],
        out_specs=[
            pl.BlockSpec((BM, H), lambda p, i: (p * i, 0)),
            pl.BlockSpec((1, H), lambda p, i: (0, 0)),
        ],
        out_shape=[
            jax.ShapeDtypeStruct((NP, H), jnp.float32),
            jax.ShapeDtypeStruct((1, H), jnp.float32),
        ],
        scratch_shapes=[
            pltpu.VMEM((NP, H), jnp.float32),
            pltpu.SMEM((2,), jnp.float32),
        ],
    )(acc, u, dinv1, b.reshape(1, H), gw.reshape(1, H), gb.reshape(1, H), wn)


def _fused3_body(acc_ref, u_ref, dinv_ref, b_ref, gw_ref, gb_ref, t1_ref,
                 t2_ref, wjk_ref, bjk_ref, wf1_ref, bf1_ref, wf2_ref, bf2_ref,
                 out_ref, ybuf, sums_ref):
    i = pl.program_id(0)

    @pl.when(i < NB)
    def _():
        y = (acc_ref[0] + acc_ref[1] - u_ref[...]) * dinv_ref[...] + b_ref[...]
        ybuf[...] = y
        _stats_accum(i, y, sums_ref)

    @pl.when(i == NB)
    def _():
        m, std = _mstd(sums_ref)
        r0 = LAST % BM
        t3 = jnp.tanh((ybuf[r0:r0 + 1, :] - m) / (std + EPS) * gw_ref[...]
                      + gb_ref[...])
        cat = jnp.concatenate([t1_ref[...], t2_ref[...], t3], axis=1)
        h = jnp.dot(cat, wjk_ref[...], preferred_element_type=jnp.float32) \
            + bjk_ref[...]
        h = jnp.tanh(jnp.dot(h, wf1_ref[...],
                             preferred_element_type=jnp.float32) + bf1_ref[...])
        out_ref[...] = jnp.dot(h, wf2_ref[...],
                               preferred_element_type=jnp.float32) \
            + bf2_ref[...]


def _fused3(acc, u, dinv1, b, gw, gb, t1, t2, wjk, bjk, wf1, bf1, wf2p, bf2p):
    blk = lambda i: (jnp.minimum(i, NB - 1), 0)
    return pl.pallas_call(
        _fused3_body,
        grid=(NB + 1,),
        in_specs=[
            pl.BlockSpec((NC, BM, H), lambda i: (0,) + blk(i)),
            pl.BlockSpec((BM, H), blk),
            pl.BlockSpec((BM, 1), blk),
            pl.BlockSpec((1, H), lambda i: (0, 0)),
            pl.BlockSpec((1, H), lambda i: (0, 0)),
            pl.BlockSpec((1, H), lambda i: (0, 0)),
            pl.BlockSpec((1, H), lambda i: (0, 0)),
            pl.BlockSpec((1, H), lambda i: (0, 0)),
            pl.BlockSpec((3 * H, H), lambda i: (0, 0)),
            pl.BlockSpec((1, H), lambda i: (0, 0)),
            pl.BlockSpec((H, HH), lambda i: (0, 0)),
            pl.BlockSpec((1, HH), lambda i: (0, 0)),
            pl.BlockSpec((HH, H), lambda i: (0, 0)),
            pl.BlockSpec((1, H), lambda i: (0, 0)),
        ],
        out_specs=pl.BlockSpec((1, H), lambda i: (0, 0)),
        out_shape=jax.ShapeDtypeStruct((1, H), jnp.float32),
        scratch_shapes=[
            pltpu.VMEM((BM, H), jnp.float32),
            pltpu.SMEM((2,), jnp.float32),
        ],
    )(acc, u, dinv1, b.reshape(1, H), gw.reshape(1, H), gb.reshape(1, H),
      t1, t2, wjk, bjk.reshape(1, H), wf1, bf1.reshape(1, HH), wf2p, bf2p)


def kernel(x, edge_index, W1, b1, W2, b2, W3, b3, g1w, g1b, g2w, g2b, g3w, g3b,
           Wjk, bjk, Wf1, bf1, Wf2, bf2):
    row = edge_index[0]
    col = edge_index[1]
    xp = jnp.pad(x, ((0, NP - N), (0, 0)))

    degp = _sc_deg(col)
    dinv1 = lax.rsqrt(degp[0] + degp[1] + 1.0).reshape(NP, 1)

    u = _mm_scale(xp, W1, dinv1)
    acc = _sc_edge(u, row, col).reshape(NC, NP, H)
    u2, t1 = _fused(acc, u, dinv1, b1, g1w, g1b, W2)
    acc = _sc_edge(u2, row, col).reshape(NC, NP, H)
    u3, t2 = _fused(acc, u2, dinv1, b2, g2w, g2b, W3)
    acc = _sc_edge(u3, row, col).reshape(NC, NP, H)

    wf2p = jnp.pad(Wf2, ((0, 0), (0, H - 1)))
    bf2p = jnp.pad(bf2.reshape(1, 1), ((0, 0), (0, H - 1)))
    out = _fused3(acc, u3, dinv1, b3, g3w, g3b, t1, t2, Wjk, bjk, Wf1, bf1,
                  wf2p, bf2p)
    return out[0, :1]


# speedup vs baseline: 20.5959x; 1.1545x over previous
import functools

import jax
import jax.numpy as jnp
from jax import lax
from jax.experimental import pallas as pl
from jax.experimental.pallas import tpu as pltpu
from jax.experimental.pallas import tpu_sc as plsc

N, E, H = 10000, 320000, 128
HH = H // 2
NC, NS, L = 2, 16, 16
NP = 10240
RPT = NP // NS
G = 80
BM = 256
NB = NP // BM
LAST = N - 1
EPS = 1e-5

_mesh = plsc.VectorSubcoreMesh(core_axis_name="c", subcore_axis_name="s")


def _sc_deg_body(col_hbm, degp_hbm, colbuf, cchunk, onesbuf, zbuf, deg_sp, sem):
    del sem
    c = lax.axis_index("c")
    s = lax.axis_index("s")
    ec = E // (NC * NS)
    for k in range(RPT // L):
        zbuf[pl.ds(k * L, L)] = jnp.zeros((L,), jnp.float32)
    for k in range(G // L):
        onesbuf[pl.ds(k * L, L)] = jnp.ones((L,), jnp.float32)
    pltpu.sync_copy(zbuf, deg_sp.at[pl.ds(s * RPT, RPT)])
    base = (c * NS + s) * ec
    pltpu.sync_copy(col_hbm.at[pl.ds(base, ec)], colbuf)
    plsc.subcore_barrier()

    def chunk(i, carry):
        for j in range(G // L):
            cchunk[pl.ds(j * L, L)] = colbuf[pl.ds(i * G + j * L, L)]
        pltpu.sync_copy(onesbuf, deg_sp.at[cchunk], add=True)
        return carry

    lax.fori_loop(0, ec // G, chunk, 0)
    plsc.subcore_barrier()
    pltpu.sync_copy(deg_sp.at[pl.ds(s * RPT, RPT)],
                    degp_hbm.at[c, pl.ds(s * RPT, RPT)])


_sc_deg = pl.kernel(
    _sc_deg_body,
    out_type=jax.ShapeDtypeStruct((NC, NP), jnp.float32),
    mesh=_mesh,
    scratch_types=[
        pltpu.VMEM((E // (NC * NS),), jnp.int32),
        pltpu.VMEM((G,), jnp.int32),
        pltpu.VMEM((G,), jnp.float32),
        pltpu.VMEM((RPT,), jnp.float32),
        pltpu.VMEM_SHARED((NP,), jnp.float32),
        pltpu.SemaphoreType.DMA,
    ],
)


NBUF = 3
GE = 64
ET = E // (NC * NS)
NCHE = ET // GE
TAIL = ET - NCHE * GE


def _sc_edge_body(u_hbm, row_hbm, col_hbm, out_hbm, rowbuf, colbuf, cchunk,
                  gbuf, tcc, tgb, acc_sp, gsem, ssem):
    c = lax.axis_index("c")
    s = lax.axis_index("s")
    gsems = [gsem.at[k] for k in range(NBUF)]
    ssems = [ssem.at[k] for k in range(NBUF)]
    et = ET
    pltpu.sync_copy(u_hbm.at[pl.ds(s * RPT, RPT)],
                    acc_sp.at[pl.ds(s * RPT, RPT)])
    ebase = (c * NS + s) * et
    pltpu.sync_copy(row_hbm.at[pl.ds(ebase, et)], rowbuf)
    pltpu.sync_copy(col_hbm.at[pl.ds(ebase, et)], colbuf)
    plsc.subcore_barrier()

    def scat_desc(k):
        return pltpu.make_async_copy(gbuf.at[k], acc_sp.at[cchunk.at[k]],
                                     ssems[k])

    def start_gather(i, k):
        for j in range(GE // L):
            cchunk[k, pl.ds(j * L, L)] = colbuf[pl.ds(i * GE + j * L, L)]
        return pltpu.async_copy(
            u_hbm.at[rowbuf.at[pl.ds(i * GE, GE)]], gbuf.at[k], gsems[k])

    ngrp, rem = divmod(NCHE, NBUF)

    def group(q, carry):
        gds = []
        for k in range(NBUF):
            @pl.when(q > 0)
            def _():
                scat_desc(k).wait()
            gds.append(start_gather(q * NBUF + k, k))
        for k in range(NBUF):
            gds[k].wait()
            pltpu.async_copy(gbuf.at[k], acc_sp.at[cchunk.at[k]], ssems[k],
                             add=True)
        return carry

    lax.fori_loop(0, ngrp, group, 0)
    for k in range(rem):
        if ngrp > 0:
            scat_desc(k).wait()
        gd = start_gather(ngrp * NBUF + k, k)
        gd.wait()
        pltpu.async_copy(gbuf.at[k], acc_sp.at[cchunk.at[k]], ssems[k],
                         add=True)
    for k in range(NBUF):
        if ngrp > 0 or k < rem:
            scat_desc(k).wait()
    if TAIL:
        tcc[...] = colbuf[pl.ds(NCHE * GE, TAIL)]
        pltpu.async_copy(u_hbm.at[rowbuf.at[pl.ds(NCHE * GE, TAIL)]],
                         tgb, gsems[0]).wait()
        pltpu.sync_copy(tgb, acc_sp.at[tcc], add=True)
    plsc.subcore_barrier()
    pltpu.sync_copy(acc_sp.at[pl.ds(s * RPT, RPT)],
                    out_hbm.at[pl.ds(c * NP + s * RPT, RPT)])


_sc_edge = pl.kernel(
    _sc_edge_body,
    out_type=jax.ShapeDtypeStruct((NC * NP, H), jnp.float32),
    mesh=_mesh,
    scratch_types=[
        pltpu.VMEM((ET,), jnp.int32),
        pltpu.VMEM((ET,), jnp.int32),
        pltpu.VMEM((NBUF, GE), jnp.int32),
        pltpu.VMEM((NBUF, GE, H), jnp.float32),
        pltpu.VMEM((TAIL,), jnp.int32),
        pltpu.VMEM((TAIL, H), jnp.float32),
        pltpu.VMEM_SHARED((NP, H), jnp.float32),
        pltpu.SemaphoreType.DMA((NBUF,)),
        pltpu.SemaphoreType.DMA((NBUF,)),
    ],
)


def _mm_scale_body(x_ref, w_ref, dinv_ref, out_ref):
    out_ref[...] = jnp.dot(x_ref[...], w_ref[...],
                           preferred_element_type=jnp.float32) * dinv_ref[...]


def _mm_scale(xp, w, dinv1):
    return pl.pallas_call(
        _mm_scale_body,
        grid=(NB,),
        in_specs=[
            pl.BlockSpec((BM, H), lambda i: (i, 0)),
            pl.BlockSpec((H, H), lambda i: (0, 0)),
            pl.BlockSpec((BM, 1), lambda i: (i, 0)),
        ],
        out_specs=pl.BlockSpec((BM, H), lambda i: (i, 0)),
        out_shape=jax.ShapeDtypeStruct((NP, H), jnp.float32),
    )(xp, w, dinv1)


NH = float(N * H)


def _stats_accum(i, y, sums_ref):
    ridx = lax.broadcasted_iota(jnp.int32, (BM, H), 0) + i * BM
    ym = jnp.where(ridx < N, y, 0.0)

    @pl.when(i == 0)
    def _():
        sums_ref[0] = 0.0
        sums_ref[1] = 0.0

    sums_ref[0] += jnp.sum(ym)
    sums_ref[1] += jnp.sum(ym * ym)


def _mstd(sums_ref):
    m = sums_ref[0] / NH
    std = jnp.sqrt(jnp.maximum(sums_ref[1] / NH - m * m, 0.0))
    return m, std


def _fused_body(acc_ref, u_ref, dinv_ref, b_ref, gw_ref, gb_ref, w_ref,
                out_ref, tl_ref, ybuf, sums_ref):
    p = pl.program_id(0)
    i = pl.program_id(1)

    @pl.when(p == 0)
    def _():
        y = (acc_ref[0] + acc_ref[1] - u_ref[...]) * dinv_ref[...] + b_ref[...]
        ybuf[pl.ds(i * BM, BM), :] = y
        _stats_accum(i, y, sums_ref)

    @pl.when(p == 1)
    def _():
        m, std = _mstd(sums_ref)
        t = jnp.tanh((ybuf[pl.ds(i * BM, BM), :] - m) / (std + EPS)
                     * gw_ref[...] + gb_ref[...])
        out_ref[...] = jnp.dot(
            t, w_ref[...], preferred_element_type=jnp.float32) * dinv_ref[...]
        tl_ref[...] = t[LAST % BM:LAST % BM + 1, :]


def _fused(acc, u, dinv1, b, gw, gb, wn):
    return pl.pallas_call(
        _fused_body,
        grid=(2, NB),
        in_specs=[
            pl.BlockSpec((NC, BM, H), lambda p, i: (0, (1 - p) * i, 0)),
            pl.BlockSpec((BM, H), lambda p, i: ((1 - p) * i, 0)),
            pl.BlockSpec((BM, 1), lambda p, i: (i, 0)),
            pl.BlockSpec((1, H), lambda p, i: (0, 0)),
            pl.BlockSpec((1, H), lambda p, i: (0, 0)),
            pl.BlockSpec((1, H), lambda p, i: (0, 0)),
            pl.BlockSpec((H, H), lambda p, i: (0, 0)),
        ],
        out_specs=[
            pl.BlockSpec((BM, H), lambda p, i: (p * i, 0)),
            pl.BlockSpec((1, H), lambda p, i: (0, 0)),
        ],
        out_shape=[
            jax.ShapeDtypeStruct((NP, H), jnp.float32),
            jax.ShapeDtypeStruct((1, H), jnp.float32),
        ],
        scratch_shapes=[
            pltpu.VMEM((NP, H), jnp.float32),
            pltpu.SMEM((2,), jnp.float32),
        ],
    )(acc, u, dinv1, b.reshape(1, H), gw.reshape(1, H), gb.reshape(1, H), wn)


def _fused3_body(acc_ref, u_ref, dinv_ref, b_ref, gw_ref, gb_ref, t1_ref,
                 t2_ref, wjk_ref, bjk_ref, wf1_ref, bf1_ref, wf2_ref, bf2_ref,
                 out_ref, ybuf, sums_ref):
    i = pl.program_id(0)

    @pl.when(i < NB)
    def _():
        y = (acc_ref[0] + acc_ref[1] - u_ref[...]) * dinv_ref[...] + b_ref[...]
        ybuf[...] = y
        _stats_accum(i, y, sums_ref)

    @pl.when(i == NB)
    def _():
        m, std = _mstd(sums_ref)
        r0 = LAST % BM
        t3 = jnp.tanh((ybuf[r0:r0 + 1, :] - m) / (std + EPS) * gw_ref[...]
                      + gb_ref[...])
        cat = jnp.concatenate([t1_ref[...], t2_ref[...], t3], axis=1)
        h = jnp.dot(cat, wjk_ref[...], preferred_element_type=jnp.float32) \
            + bjk_ref[...]
        h = jnp.tanh(jnp.dot(h, wf1_ref[...],
                             preferred_element_type=jnp.float32) + bf1_ref[...])
        out_ref[...] = jnp.dot(h, wf2_ref[...],
                               preferred_element_type=jnp.float32) \
            + bf2_ref[...]


def _fused3(acc, u, dinv1, b, gw, gb, t1, t2, wjk, bjk, wf1, bf1, wf2p, bf2p):
    blk = lambda i: (jnp.minimum(i, NB - 1), 0)
    return pl.pallas_call(
        _fused3_body,
        grid=(NB + 1,),
        in_specs=[
            pl.BlockSpec((NC, BM, H), lambda i: (0,) + blk(i)),
            pl.BlockSpec((BM, H), blk),
            pl.BlockSpec((BM, 1), blk),
            pl.BlockSpec((1, H), lambda i: (0, 0)),
            pl.BlockSpec((1, H), lambda i: (0, 0)),
            pl.BlockSpec((1, H), lambda i: (0, 0)),
            pl.BlockSpec((1, H), lambda i: (0, 0)),
            pl.BlockSpec((1, H), lambda i: (0, 0)),
            pl.BlockSpec((3 * H, H), lambda i: (0, 0)),
            pl.BlockSpec((1, H), lambda i: (0, 0)),
            pl.BlockSpec((H, HH), lambda i: (0, 0)),
            pl.BlockSpec((1, HH), lambda i: (0, 0)),
            pl.BlockSpec((HH, H), lambda i: (0, 0)),
            pl.BlockSpec((1, H), lambda i: (0, 0)),
        ],
        out_specs=pl.BlockSpec((1, H), lambda i: (0, 0)),
        out_shape=jax.ShapeDtypeStruct((1, H), jnp.float32),
        scratch_shapes=[
            pltpu.VMEM((BM, H), jnp.float32),
            pltpu.SMEM((2,), jnp.float32),
        ],
    )(acc, u, dinv1, b.reshape(1, H), gw.reshape(1, H), gb.reshape(1, H),
      t1, t2, wjk, bjk.reshape(1, H), wf1, bf1.reshape(1, HH), wf2p, bf2p)


def kernel(x, edge_index, W1, b1, W2, b2, W3, b3, g1w, g1b, g2w, g2b, g3w, g3b,
           Wjk, bjk, Wf1, bf1, Wf2, bf2):
    row = edge_index[0]
    col = edge_index[1]
    xp = jnp.pad(x, ((0, NP - N), (0, 0)))

    degp = _sc_deg(col)
    dinv1 = lax.rsqrt(degp[0] + degp[1] + 1.0).reshape(NP, 1)

    u = _mm_scale(xp, W1, dinv1)
    acc = _sc_edge(u, row, col).reshape(NC, NP, H)
    u2, t1 = _fused(acc, u, dinv1, b1, g1w, g1b, W2)
    acc = _sc_edge(u2, row, col).reshape(NC, NP, H)
    u3, t2 = _fused(acc, u2, dinv1, b2, g2w, g2b, W3)
    acc = _sc_edge(u3, row, col).reshape(NC, NP, H)

    wf2p = jnp.pad(Wf2, ((0, 0), (0, H - 1)))
    bf2p = jnp.pad(bf2.reshape(1, 1), ((0, 0), (0, H - 1)))
    out = _fused3(acc, u3, dinv1, b3, g3w, g3b, t1, t2, Wjk, bjk, Wf1, bf1,
                  wf2p, bf2p)
    return out[0, :1]
